# baseline (speedup 1.0000x reference)
import functools

import jax
import jax.numpy as jnp
import numpy as np
from jax import lax
from jax.experimental import pallas as pl
from jax.experimental.pallas import tpu as pltpu
from jax.experimental.pallas import tpu_sc as plsc

_HIDDEN = 128
_TABLE_ROWS = 1000
_IDX_CHUNK = 128
_LANES = 16


def _sc_gather(table, n_rows, nw):
    rows_per_w = n_rows // nw
    n_ch = rows_per_w // _IDX_CHUNK
    step = float(np.float32(1.0) / np.float32(n_rows - 1))
    scale = float(_TABLE_ROWS)
    hi = float(_TABLE_ROWS - 1)
    mesh = plsc.VectorSubcoreMesh(core_axis_name="c", subcore_axis_name="s")

    @functools.partial(
        pl.kernel,
        mesh=mesh,
        out_type=jax.ShapeDtypeStruct((n_rows, _HIDDEN), jnp.float32),
        scratch_types=[
            pltpu.VMEM((n_ch, _IDX_CHUNK), jnp.int32),
            pltpu.VMEM((rows_per_w, _HIDDEN), jnp.float32),
            pltpu.SemaphoreType.DMA,
            pltpu.SemaphoreType.DMA,
        ],
    )
    def gather_kernel(table_hbm, out_hbm, idx_v, rows_v, gsem, ssem):
        num_cores = lax.axis_size("c")
        wid = lax.axis_index("s") * num_cores + lax.axis_index("c")
        base = wid * rows_per_w
        lane = lax.iota(jnp.int32, _LANES)
        for c in range(n_ch):
            for w in range(_IDX_CHUNK // _LANES):
                r = lane + (base + c * _IDX_CHUNK + w * _LANES)
                u = r.astype(jnp.float32) * step * scale
                idx_v[c, pl.ds(w * _LANES, _LANES)] = jnp.clip(u, 0.0, hi).astype(
                    jnp.int32
                )
        gathers = [
            pltpu.async_copy(
                table_hbm.at[idx_v.at[c]],
                rows_v.at[pl.ds(c * _IDX_CHUNK, _IDX_CHUNK)],
                gsem,
            )
            for c in range(n_ch)
        ]
        scatters = []
        for c in range(n_ch):
            gathers[c].wait()
            scatters.append(
                pltpu.async_copy(
                    rows_v.at[pl.ds(c * _IDX_CHUNK, _IDX_CHUNK)],
                    out_hbm.at[pl.ds(base + c * _IDX_CHUNK, _IDX_CHUNK)],
                    ssem,
                )
            )
        for s in scatters:
            s.wait()

    return gather_kernel(table)


def _add_body(f_ref, p_ref, o_ref):
    pos = p_ref[...]
    o_ref[...] = f_ref[...] + pos[None, :, :]


def _tc_add(features, pos, block_rows):
    n, t, d = features.shape
    ft = jnp.transpose(features, (1, 0, 2))
    grid = (n // block_rows, t)
    out_t = pl.pallas_call(
        _add_body,
        grid=grid,
        in_specs=[
            pl.BlockSpec((1, block_rows, d), lambda j, i: (i, j, 0)),
            pl.BlockSpec((block_rows, d), lambda j, i: (j, 0)),
        ],
        out_specs=pl.BlockSpec((1, block_rows, d), lambda j, i: (i, j, 0)),
        out_shape=jax.ShapeDtypeStruct((t, n, d), features.dtype),
    )(ft, pos)
    return jnp.transpose(out_t, (1, 0, 2))


def kernel(features, temporal_embedding):
    n = features.shape[0]
    info = plsc.get_sparse_core_info()
    nw = info.num_cores * info.num_subcores
    pos = _sc_gather(temporal_embedding, n, nw)
    return _tc_add(features, pos, block_rows=n)
